# baseline (speedup 1.0000x reference)
import jax
import jax.numpy as jnp
from jax import lax
from jax.experimental import pallas as pl
from jax.experimental.pallas import tpu as pltpu
from jax.experimental.pallas import tpu_sc as plsc

MAX_REL = 128
NUM_HEADS = 16
SEQ_LEN = 2048
NPHASE = 16
RWIDTH = 34 * 128

NUM_CORES = 2
NUM_SUBCORES = 16
NUM_WORKERS = NUM_CORES * NUM_SUBCORES
NUM_SLABS = NUM_HEADS * (SEQ_LEN // 8)
SLABS_PER_WORKER = NUM_SLABS // NUM_WORKERS
INFLIGHT = 8


def _bias_body(tbl_hbm, out_hbm, sem):
    cid = lax.axis_index("c")
    sid = lax.axis_index("s")
    wid = sid * NUM_CORES + cid
    head = wid // 2
    base_a = (wid % 2) * SLABS_PER_WORKER

    def slab_copy(k):
        a = base_a + k
        p = lax.rem(a, NPHASE)
        q = a // NPHASE
        col = 128 * (16 - q)
        return pltpu.make_async_copy(
            tbl_hbm.at[head, p, :, pl.ds(pl.multiple_of(col, 128), SEQ_LEN)],
            out_hbm.at[head, pl.ds(pl.multiple_of(8 * a, 8), 8), :],
            sem,
        )

    for k in range(INFLIGHT):
        slab_copy(k).start()

    def body(k, carry):
        slab_copy(k).start()
        slab_copy(k - INFLIGHT).wait()
        return carry

    lax.fori_loop(INFLIGHT, SLABS_PER_WORKER, body, 0)

    for k in range(SLABS_PER_WORKER - INFLIGHT, SLABS_PER_WORKER):
        slab_copy(k).wait()


@jax.jit
def _bias_sc(tbl):
    mesh = plsc.VectorSubcoreMesh(core_axis_name="c", subcore_axis_name="s")
    return pl.kernel(
        _bias_body,
        out_type=jax.ShapeDtypeStruct((NUM_HEADS, SEQ_LEN, SEQ_LEN), jnp.float32),
        mesh=mesh,
        scratch_types=[pltpu.SemaphoreType.DMA],
    )(tbl)


def kernel(seq_len, W):
    del seq_len
    pad = 136
    lo = pad + SEQ_LEN - 1 - MAX_REL
    hi = RWIDTH + pad - lo - (2 * MAX_REL + 1) + 8
    fp = jnp.concatenate(
        [
            jnp.broadcast_to(W[:1], (lo, NUM_HEADS)),
            W,
            jnp.broadcast_to(W[-1:], (hi, NUM_HEADS)),
        ],
        axis=0,
    )
    rows = []
    for p in range(NPHASE):
        v = (8 * p + 1) % 128
        for r in range(8):
            rows.append(fp[pad - r - v : pad - r - v + RWIDTH])
    tbl = jnp.stack(rows, axis=0)
    tbl = jnp.transpose(tbl, (2, 0, 1)).reshape(NUM_HEADS, NPHASE, 8, RWIDTH)
    return _bias_sc(tbl)


# speedup vs baseline: 61.1104x; 61.1104x over previous
import jax
import jax.numpy as jnp
from jax import lax
from jax.experimental import pallas as pl
from jax.experimental.pallas import tpu as pltpu
from jax.experimental.pallas import tpu_sc as plsc

MAX_REL = 128
NUM_HEADS = 16
SEQ_LEN = 2048
NPHASE = 16
NTILE = SEQ_LEN // 128
BAND_W0 = 14
BAND_TILES = 5
CONST_TILES = 13

NUM_CORES = 2
NUM_SUBCORES = 16
PHASES_PER_WORKER = 8
INFLIGHT_SLABS = 4


def _bias_body(band_hbm, const_hbm, out_hbm, band_v, const_v, sem):
    cid = lax.axis_index("c")
    sid = lax.axis_index("s")
    wid = sid * NUM_CORES + cid
    head = wid // 2
    pbase = (wid % 2) * PHASES_PER_WORKER

    for e in range(PHASES_PER_WORKER):
        pltpu.sync_copy(band_hbm.at[head, pbase + e], band_v.at[e])
    pltpu.sync_copy(const_hbm.at[head], const_v)

    def slab_dmas(q, e):
        a = 16 * q + pbase + e
        row = pl.ds(pl.multiple_of(8 * a, 8), 8)
        tb = min(max(q - 1, 0), CONST_TILES)
        widx = tb + NTILE - q - BAND_W0
        dmas = []
        if tb > 0:
            dmas.append(pltpu.make_async_copy(
                const_v.at[0, :, pl.ds(0, 128 * tb)],
                out_hbm.at[head, row, pl.ds(0, 128 * tb)],
                sem,
            ))
        dmas.append(pltpu.make_async_copy(
            band_v.at[e, :, pl.ds(128 * widx, 384)],
            out_hbm.at[head, row, pl.ds(128 * tb, 384)],
            sem,
        ))
        if tb < CONST_TILES:
            n = CONST_TILES - tb
            dmas.append(pltpu.make_async_copy(
                const_v.at[1, :, pl.ds(0, 128 * n)],
                out_hbm.at[head, row, pl.ds(128 * (tb + 3), 128 * n)],
                sem,
            ))
        return dmas

    for q in range(NTILE):
        def issue(e):
            for d in slab_dmas(q, e):
                d.start()

        def drain(e):
            for d in slab_dmas(q, e):
                d.wait()

        def body(e, carry):
            issue(e)

            @pl.when(e >= INFLIGHT_SLABS)
            def _():
                drain(e - INFLIGHT_SLABS)

            return carry

        lax.fori_loop(0, PHASES_PER_WORKER, body, 0, unroll=2)
        for e in range(PHASES_PER_WORKER - INFLIGHT_SLABS, PHASES_PER_WORKER):
            drain(e)


@jax.jit
def _bias_sc(band, const):
    mesh = plsc.VectorSubcoreMesh(core_axis_name="c", subcore_axis_name="s")
    return pl.kernel(
        _bias_body,
        out_type=jax.ShapeDtypeStruct((NUM_HEADS, SEQ_LEN, SEQ_LEN), jnp.float32),
        mesh=mesh,
        scratch_types=[
            pltpu.VMEM((PHASES_PER_WORKER, 8, 128 * BAND_TILES), jnp.float32),
            pltpu.VMEM((2, 8, 128 * CONST_TILES), jnp.float32),
            pltpu.SemaphoreType.DMA,
        ],
    )(band, const)


def kernel(seq_len, W):
    del seq_len
    pad = 136
    lo = pad + SEQ_LEN - 1 - MAX_REL
    width = 128 * BAND_TILES + 1792
    hi = width + pad - lo - (2 * MAX_REL + 1) + 8
    fp = jnp.concatenate(
        [
            jnp.broadcast_to(W[:1], (lo, NUM_HEADS)),
            W,
            jnp.broadcast_to(W[-1:], (hi, NUM_HEADS)),
        ],
        axis=0,
    )
    rows = []
    for p in range(NPHASE):
        v = 8 * p + 1
        for r in range(8):
            s = pad + 1792 - r - v
            rows.append(fp[s : s + 128 * BAND_TILES])
    band = jnp.stack(rows, 0)
    band = jnp.transpose(band, (2, 0, 1)).reshape(
        NUM_HEADS, NPHASE, 8, 128 * BAND_TILES
    )
    const = jnp.broadcast_to(
        jnp.stack([W[0], W[-1]], 0).T[:, :, None, None],
        (NUM_HEADS, 2, 8, 128 * CONST_TILES),
    )
    return _bias_sc(band, const + jnp.zeros_like(const))


# speedup vs baseline: 72.1385x; 1.1805x over previous
import jax
import jax.numpy as jnp
from jax import lax
from jax.experimental import pallas as pl
from jax.experimental.pallas import tpu as pltpu
from jax.experimental.pallas import tpu_sc as plsc

MAX_REL = 128
NUM_HEADS = 16
SEQ_LEN = 2048
NPHASE = 16
NTILE = SEQ_LEN // 128
BAND_W0 = 14
BAND_TILES = 5
CONST_TILES = 13

NUM_CORES = 2
NUM_SUBCORES = 16
PHASES_PER_WORKER = 8
INFLIGHT_SLABS = 4


def _bias_body(band_hbm, const_hbm, out_hbm, band_v, const_v, sem):
    cid = lax.axis_index("c")
    sid = lax.axis_index("s")
    wid = sid * NUM_CORES + cid
    head = wid // 2
    pbase = (wid % 2) * PHASES_PER_WORKER

    for e in range(PHASES_PER_WORKER):
        pltpu.sync_copy(band_hbm.at[head, pbase + e], band_v.at[e])
    pltpu.sync_copy(const_hbm.at[head], const_v)

    def slab_dmas(q, e):
        a = 16 * q + pbase + e
        row = pl.ds(pl.multiple_of(8 * a, 8), 8)
        tb = min(max(q - 1, 0), CONST_TILES)
        widx = tb + NTILE - q - BAND_W0
        dmas = []
        if tb > 0:
            dmas.append(pltpu.make_async_copy(
                const_v.at[0, :, pl.ds(0, 128 * tb)],
                out_hbm.at[head, row, pl.ds(0, 128 * tb)],
                sem,
            ))
        dmas.append(pltpu.make_async_copy(
            band_v.at[e, :, pl.ds(128 * widx, 384)],
            out_hbm.at[head, row, pl.ds(128 * tb, 384)],
            sem,
        ))
        if tb < CONST_TILES:
            n = CONST_TILES - tb
            dmas.append(pltpu.make_async_copy(
                const_v.at[1, :, pl.ds(0, 128 * n)],
                out_hbm.at[head, row, pl.ds(128 * (tb + 3), 128 * n)],
                sem,
            ))
        return dmas

    for q in range(NTILE):
        def issue(e):
            for d in slab_dmas(q, e):
                d.start()

        def drain(e):
            for d in slab_dmas(q, e):
                d.wait()

        def body(e, carry):
            issue(e)

            @pl.when(e >= INFLIGHT_SLABS)
            def _():
                drain(e - INFLIGHT_SLABS)

            return carry

        lax.fori_loop(0, PHASES_PER_WORKER, body, 0, unroll=2)
        for e in range(PHASES_PER_WORKER - INFLIGHT_SLABS, PHASES_PER_WORKER):
            drain(e)


@jax.jit
def _bias_sc(band, const):
    mesh = plsc.VectorSubcoreMesh(core_axis_name="c", subcore_axis_name="s")
    return pl.kernel(
        _bias_body,
        out_type=jax.ShapeDtypeStruct((NUM_HEADS, SEQ_LEN, SEQ_LEN), jnp.float32),
        mesh=mesh,
        scratch_types=[
            pltpu.VMEM((PHASES_PER_WORKER, 8, 128 * BAND_TILES), jnp.float32),
            pltpu.VMEM((2, 8, 128 * CONST_TILES), jnp.float32),
            pltpu.SemaphoreType.DMA,
        ],
    )(band, const)


def kernel(seq_len, W):
    del seq_len
    pad = 136
    lo = pad + SEQ_LEN - 1 - MAX_REL
    wt = W.T
    fpt = jnp.concatenate(
        [
            jnp.broadcast_to(wt[:, :1], (NUM_HEADS, lo)),
            wt,
            jnp.broadcast_to(wt[:, -1:], (NUM_HEADS, 264)),
        ],
        axis=1,
    )
    s8 = jnp.stack(
        [fpt[:, pad + 1664 - r : pad + 2432 - r] for r in range(8)], axis=1
    )
    band = jnp.stack(
        [s8[:, :, 127 - 8 * p : 767 - 8 * p] for p in range(NPHASE)], axis=1
    )
    const = jnp.broadcast_to(
        jnp.stack([W[0], W[-1]], 0).T[:, :, None, None],
        (NUM_HEADS, 2, 8, 128 * CONST_TILES),
    )
    return _bias_sc(band, const + jnp.zeros_like(const))


# speedup vs baseline: 72.4855x; 1.0048x over previous
import jax
import jax.numpy as jnp
from jax import lax
from jax.experimental import pallas as pl
from jax.experimental.pallas import tpu as pltpu
from jax.experimental.pallas import tpu_sc as plsc

MAX_REL = 128
NUM_HEADS = 16
SEQ_LEN = 2048
NPHASE = 16
NTILE = SEQ_LEN // 128
BAND_W0 = 14
BAND_TILES = 5
CONST_TILES = 13

NUM_CORES = 2
NUM_SUBCORES = 16
PHASES_PER_WORKER = 8
INFLIGHT_SLABS = 4


def _bias_body(band_hbm, const_hbm, out_hbm, band_v, const_v, sem):
    cid = lax.axis_index("c")
    sid = lax.axis_index("s")
    wid = sid * NUM_CORES + cid
    head = wid // 2
    pbase = (wid % 2) * PHASES_PER_WORKER

    for e in range(PHASES_PER_WORKER):
        pltpu.sync_copy(band_hbm.at[head, pbase + e], band_v.at[e])
    pltpu.sync_copy(const_hbm.at[head], const_v)

    def slab_dmas(q, e):
        a = 16 * q + pbase + e
        row = pl.ds(pl.multiple_of(8 * a, 8), 8)
        tb = min(max(q - 1, 0), CONST_TILES)
        widx = tb + NTILE - q - BAND_W0
        dmas = []
        if tb > 0:
            dmas.append(pltpu.make_async_copy(
                const_v.at[0, :, pl.ds(0, 128 * tb)],
                out_hbm.at[head, row, pl.ds(0, 128 * tb)],
                sem,
            ))
        dmas.append(pltpu.make_async_copy(
            band_v.at[e, :, pl.ds(128 * widx, 384)],
            out_hbm.at[head, row, pl.ds(128 * tb, 384)],
            sem,
        ))
        if tb < CONST_TILES:
            n = CONST_TILES - tb
            dmas.append(pltpu.make_async_copy(
                const_v.at[1, :, pl.ds(0, 128 * n)],
                out_hbm.at[head, row, pl.ds(128 * (tb + 3), 128 * n)],
                sem,
            ))
        return dmas

    K = INFLIGHT_SLABS

    def drain(q, e):
        for d in slab_dmas(q, e):
            d.wait()

    for q in range(NTILE):
        def body(e, carry, q=q):
            for d in slab_dmas(q, e):
                d.start()

            @pl.when(e >= K)
            def _():
                drain(q, e - K)

            if q > 0:

                @pl.when(e < K)
                def _():
                    drain(q - 1, e + PHASES_PER_WORKER - K)

            return carry

        lax.fori_loop(0, PHASES_PER_WORKER, body, 0, unroll=2)
    for e in range(PHASES_PER_WORKER - K, PHASES_PER_WORKER):
        drain(NTILE - 1, e)


@jax.jit
def _bias_sc(band, const):
    mesh = plsc.VectorSubcoreMesh(core_axis_name="c", subcore_axis_name="s")
    return pl.kernel(
        _bias_body,
        out_type=jax.ShapeDtypeStruct((NUM_HEADS, SEQ_LEN, SEQ_LEN), jnp.float32),
        mesh=mesh,
        scratch_types=[
            pltpu.VMEM((PHASES_PER_WORKER, 8, 128 * BAND_TILES), jnp.float32),
            pltpu.VMEM((2, 8, 128 * CONST_TILES), jnp.float32),
            pltpu.SemaphoreType.DMA,
        ],
    )(band, const)


def kernel(seq_len, W):
    del seq_len
    pad = 136
    lo = pad + SEQ_LEN - 1 - MAX_REL
    wt = W.T
    fpt = jnp.concatenate(
        [
            jnp.broadcast_to(wt[:, :1], (NUM_HEADS, lo)),
            wt,
            jnp.broadcast_to(wt[:, -1:], (NUM_HEADS, 264)),
        ],
        axis=1,
    )
    s8 = jnp.stack(
        [fpt[:, pad + 1664 - r : pad + 2432 - r] for r in range(8)], axis=1
    )
    band = jnp.stack(
        [s8[:, :, 127 - 8 * p : 767 - 8 * p] for p in range(NPHASE)], axis=1
    )
    const = jnp.broadcast_to(
        jnp.stack([W[0], W[-1]], 0).T[:, :, None, None],
        (NUM_HEADS, 2, 8, 128 * CONST_TILES),
    )
    return _bias_sc(band, const + jnp.zeros_like(const))
